# baseline (speedup 1.0000x reference)
import functools

import jax
import jax.numpy as jnp
from jax import lax
from jax.experimental import pallas as pl
from jax.experimental.pallas import tpu as pltpu
from jax.experimental.pallas import tpu_sc as plsc

N_SRC = 163842
N_ROWS = 40962
C_IN = 32
C_OUT = 64
K = 7

NC = 2
NS = 16
NW = NC * NS
ROWS_PW = 1296
N_PAD = NW * ROWS_PW

CHUNK_P = 144
CHUNK_G = 72

TC_TILE = 1536
TC_GRID = N_PAD // TC_TILE
EPS = 1e-5

_MESH = plsc.VectorSubcoreMesh(core_axis_name="c", subcore_axis_name="s")
_SC_PARAMS = pltpu.CompilerParams(use_tc_tiling_on_sc=False)


@functools.partial(
    pl.kernel,
    out_type=jax.ShapeDtypeStruct((N_PAD, C_IN), jnp.float32),
    mesh=_MESH,
    scratch_types=[
        pltpu.VMEM((CHUNK_P * K,), jnp.int32),
        pltpu.VMEM((CHUNK_P * K,), jnp.int32),
        pltpu.VMEM((CHUNK_P * K, C_IN), jnp.float32),
        pltpu.VMEM((CHUNK_P * K, C_IN), jnp.float32),
        pltpu.VMEM((CHUNK_P, C_IN), jnp.float32),
        pltpu.VMEM((CHUNK_P, C_IN), jnp.float32),
        pltpu.SemaphoreType.DMA,
        pltpu.SemaphoreType.DMA,
        pltpu.SemaphoreType.DMA,
        pltpu.SemaphoreType.DMA,
    ],
    compiler_params=_SC_PARAMS,
)
def _sc_pool(x_hbm, pidx_hbm, pooled_hbm,
             idx0, idx1, buf0, buf1, ob0, ob1, gs0, gs1, ss0, ss1):
    idxs, bufs, obs = [idx0, idx1], [buf0, buf1], [ob0, ob1]
    gsems, ssems = [gs0, gs1], [ss0, ss1]

    def run(base0, nchunk):
        pltpu.sync_copy(pidx_hbm.at[pl.ds(base0 * K, CHUNK_P * K)], idxs[0])
        gd = [pltpu.async_copy(x_hbm.at[idxs[0]], bufs[0], gsems[0]), None]
        sd = [None, None]
        for c in range(nchunk):
            b, nb = c & 1, (c + 1) & 1
            if c + 1 < nchunk:
                nbase = base0 + (c + 1) * CHUNK_P
                pltpu.sync_copy(pidx_hbm.at[pl.ds(nbase * K, CHUNK_P * K)],
                                idxs[nb])
                gd[nb] = pltpu.async_copy(x_hbm.at[idxs[nb]], bufs[nb],
                                          gsems[nb])
            gd[b].wait()
            if sd[b] is not None:
                sd[b].wait()
            buf, ob = bufs[b], obs[b]

            def row(i, carry):
                for h in range(C_IN // 16):
                    sl = pl.ds(h * 16, 16)
                    s = buf[i * K, sl]
                    for k in range(1, K):
                        s = s + buf[i * K + k, sl]
                    ob[i, sl] = s * (1.0 / K)
                return carry

            lax.fori_loop(0, CHUNK_P, row, 0)
            sd[b] = pltpu.async_copy(
                obs[b], pooled_hbm.at[pl.ds(base0 + c * CHUNK_P, CHUNK_P)],
                ssems[b])
        for d in sd:
            if d is not None:
                d.wait()

    wid = lax.axis_index("s") * NC + lax.axis_index("c")
    run(wid * ROWS_PW, ROWS_PW // CHUNK_P)


def _make_sc_gather(ch, chunk):
    @functools.partial(
        pl.kernel,
        out_type=jax.ShapeDtypeStruct((N_PAD * K, ch), jnp.float32),
        mesh=_MESH,
        scratch_types=[
            pltpu.VMEM((chunk * K,), jnp.int32),
            pltpu.VMEM((chunk * K,), jnp.int32),
            pltpu.VMEM((chunk * K, ch), jnp.float32),
            pltpu.VMEM((chunk * K, ch), jnp.float32),
            pltpu.SemaphoreType.DMA,
            pltpu.SemaphoreType.DMA,
            pltpu.SemaphoreType.DMA,
            pltpu.SemaphoreType.DMA,
        ],
        compiler_params=_SC_PARAMS,
    )
    def _sc_gather(table_hbm, idx_hbm, out_hbm,
                   idx0, idx1, buf0, buf1, gs0, gs1, ss0, ss1):
        idxs, bufs = [idx0, idx1], [buf0, buf1]
        gsems, ssems = [gs0, gs1], [ss0, ss1]

        def run(base0, nchunk):
            pltpu.sync_copy(idx_hbm.at[pl.ds(base0, chunk * K)], idxs[0])
            gd = [pltpu.async_copy(table_hbm.at[idxs[0]], bufs[0], gsems[0]),
                  None]
            sd = [None, None]
            for c in range(nchunk):
                b, nb = c & 1, (c + 1) & 1
                if c + 1 < nchunk:
                    nbase = base0 + (c + 1) * chunk * K
                    pltpu.sync_copy(idx_hbm.at[pl.ds(nbase, chunk * K)],
                                    idxs[nb])
                    if sd[nb] is not None:
                        sd[nb].wait()
                    gd[nb] = pltpu.async_copy(table_hbm.at[idxs[nb]],
                                              bufs[nb], gsems[nb])
                gd[b].wait()
                sd[b] = pltpu.async_copy(
                    bufs[b],
                    out_hbm.at[pl.ds(base0 + c * chunk * K, chunk * K)],
                    ssems[b])
            for d in sd:
                if d is not None:
                    d.wait()

        wid = lax.axis_index("s") * NC + lax.axis_index("c")
        run(wid * ROWS_PW * K, ROWS_PW // chunk)

    return _sc_gather


_sc_gather32 = _make_sc_gather(C_IN, CHUNK_P)
_sc_gather64 = _make_sc_gather(C_OUT, CHUNK_G)


def _tc_mm1_body(g1_ref, w1_ref, b1_ref, h_ref, st_ref):
    i = pl.program_id(0)
    h = lax.dot_general(
        g1_ref[...], w1_ref[...], (((1,), (1,)), ((), ())),
        preferred_element_type=jnp.float32,
    ) + b1_ref[...]
    h_ref[...] = h
    rows = i * TC_TILE + lax.broadcasted_iota(jnp.int32, (TC_TILE, 1), 0)
    hm = jnp.where(rows < N_ROWS, h, 0.0)

    @pl.when(i == 0)
    def _():
        st_ref[...] = jnp.zeros((8, 128), jnp.float32)

    st_ref[0:1, 0:C_OUT] += jnp.sum(hm, axis=0)[None, :]
    st_ref[1:2, 0:C_OUT] += jnp.sum(hm * hm, axis=0)[None, :]


def _tc_mm1(g1, w1, b1):
    return pl.pallas_call(
        _tc_mm1_body,
        grid=(TC_GRID,),
        in_specs=[
            pl.BlockSpec((TC_TILE, K * C_IN), lambda i: (i, 0)),
            pl.BlockSpec((C_OUT, K * C_IN), lambda i: (0, 0)),
            pl.BlockSpec((1, C_OUT), lambda i: (0, 0)),
        ],
        out_specs=[
            pl.BlockSpec((TC_TILE, C_OUT), lambda i: (i, 0)),
            pl.BlockSpec((8, 128), lambda i: (0, 0)),
        ],
        out_shape=[
            jax.ShapeDtypeStruct((N_PAD, C_OUT), jnp.float32),
            jax.ShapeDtypeStruct((8, 128), jnp.float32),
        ],
    )(g1, w1, b1)


def _bn_coeffs(st_ref, gamma_ref, beta_ref, width):
    inv_n = 1.0 / N_ROWS
    mu = st_ref[0:1, 0:width] * inv_n
    var = st_ref[1:2, 0:width] * inv_n - mu * mu
    a = gamma_ref[...] * lax.rsqrt(var + EPS)
    c = beta_ref[...] - a * mu
    return a, c


def _tc_mm2_body(g2_ref, w2_ref, st1_ref, ga1_ref, be1_ref, b2_ref,
                 h2_ref, st2_ref):
    i = pl.program_id(0)
    a, c = _bn_coeffs(st1_ref, ga1_ref, be1_ref, C_OUT)
    acc = jnp.zeros((TC_TILE, C_OUT), jnp.float32)
    for k in range(K):
        z = g2_ref[:, k * C_OUT:(k + 1) * C_OUT] * a + c
        z = jnp.where(z >= 0, z, 0.2 * z)
        acc = acc + lax.dot_general(
            z, w2_ref[:, k * C_OUT:(k + 1) * C_OUT],
            (((1,), (1,)), ((), ())), preferred_element_type=jnp.float32,
        )
    h2 = acc + b2_ref[...]
    h2_ref[...] = h2
    rows = i * TC_TILE + lax.broadcasted_iota(jnp.int32, (TC_TILE, 1), 0)
    hm = jnp.where(rows < N_ROWS, h2, 0.0)

    @pl.when(i == 0)
    def _():
        st2_ref[...] = jnp.zeros((8, 128), jnp.float32)

    st2_ref[0:1, 0:C_OUT] += jnp.sum(hm, axis=0)[None, :]
    st2_ref[1:2, 0:C_OUT] += jnp.sum(hm * hm, axis=0)[None, :]


def _tc_mm2(g2, w2, st1, gamma1, beta1, b2):
    return pl.pallas_call(
        _tc_mm2_body,
        grid=(TC_GRID,),
        in_specs=[
            pl.BlockSpec((TC_TILE, K * C_OUT), lambda i: (i, 0)),
            pl.BlockSpec((C_OUT, K * C_OUT), lambda i: (0, 0)),
            pl.BlockSpec((8, 128), lambda i: (0, 0)),
            pl.BlockSpec((1, C_OUT), lambda i: (0, 0)),
            pl.BlockSpec((1, C_OUT), lambda i: (0, 0)),
            pl.BlockSpec((1, C_OUT), lambda i: (0, 0)),
        ],
        out_specs=[
            pl.BlockSpec((TC_TILE, C_OUT), lambda i: (i, 0)),
            pl.BlockSpec((8, 128), lambda i: (0, 0)),
        ],
        out_shape=[
            jax.ShapeDtypeStruct((N_PAD, C_OUT), jnp.float32),
            jax.ShapeDtypeStruct((8, 128), jnp.float32),
        ],
    )(g2, w2, st1, gamma1, beta1, b2)


def _tc_bn_body(h2_ref, st2_ref, ga2_ref, be2_ref, out_ref):
    a, c = _bn_coeffs(st2_ref, ga2_ref, be2_ref, C_OUT)
    y = h2_ref[...] * a + c
    out_ref[...] = jnp.where(y >= 0, y, 0.2 * y)


def _tc_bn(h2, st2, gamma2, beta2):
    return pl.pallas_call(
        _tc_bn_body,
        grid=(TC_GRID,),
        in_specs=[
            pl.BlockSpec((TC_TILE, C_OUT), lambda i: (i, 0)),
            pl.BlockSpec((8, 128), lambda i: (0, 0)),
            pl.BlockSpec((1, C_OUT), lambda i: (0, 0)),
            pl.BlockSpec((1, C_OUT), lambda i: (0, 0)),
        ],
        out_specs=pl.BlockSpec((TC_TILE, C_OUT), lambda i: (i, 0)),
        out_shape=jax.ShapeDtypeStruct((N_ROWS, C_OUT), jnp.float32),
    )(h2, st2, gamma2, beta2)


def kernel(x, neigh_orders, pool_neigh_orders, W1, b1, gamma1, beta1,
           W2, b2, gamma2, beta2):
    pad = (N_PAD - N_ROWS) * K
    pidx = jnp.concatenate(
        [pool_neigh_orders.astype(jnp.int32), jnp.zeros((pad,), jnp.int32)])
    nidx = jnp.concatenate(
        [neigh_orders.astype(jnp.int32), jnp.zeros((pad,), jnp.int32)])

    pooled = _sc_pool(x, pidx)
    g1 = _sc_gather32(pooled, nidx).reshape(N_PAD, K * C_IN)
    h_raw, st1 = _tc_mm1(g1, W1, b1.reshape(1, C_OUT))
    g2 = _sc_gather64(h_raw, nidx).reshape(N_PAD, K * C_OUT)
    h2_raw, st2 = _tc_mm2(g2, W2, st1, gamma1.reshape(1, C_OUT),
                          beta1.reshape(1, C_OUT), b2.reshape(1, C_OUT))
    return _tc_bn(h2_raw, st2, gamma2.reshape(1, C_OUT),
                  beta2.reshape(1, C_OUT))


# speedup vs baseline: 1.0233x; 1.0233x over previous
import functools

import jax
import jax.numpy as jnp
from jax import lax
from jax.experimental import pallas as pl
from jax.experimental.pallas import tpu as pltpu
from jax.experimental.pallas import tpu_sc as plsc

N_SRC = 163842
N_ROWS = 40962
C_IN = 32
C_OUT = 64
K = 7

NC = 2
NS = 16
NW = NC * NS
ROWS_PW = 1296
N_PAD = NW * ROWS_PW

CHUNK_P = 144
CHUNK_G = 72

TC_TILE = 1536
TC_GRID = N_PAD // TC_TILE
EPS = 1e-5

_MESH = plsc.VectorSubcoreMesh(core_axis_name="c", subcore_axis_name="s")
_SC_PARAMS = pltpu.CompilerParams(use_tc_tiling_on_sc=False)


@functools.partial(
    pl.kernel,
    out_type=jax.ShapeDtypeStruct((N_PAD, C_IN), jnp.float32),
    mesh=_MESH,
    scratch_types=[
        pltpu.VMEM((CHUNK_P * K,), jnp.int32),
        pltpu.VMEM((CHUNK_P * K,), jnp.int32),
        pltpu.VMEM((CHUNK_P * K, C_IN), jnp.float32),
        pltpu.VMEM((CHUNK_P * K, C_IN), jnp.float32),
        pltpu.VMEM((CHUNK_P, C_IN), jnp.float32),
        pltpu.VMEM((CHUNK_P, C_IN), jnp.float32),
        pltpu.SemaphoreType.DMA,
        pltpu.SemaphoreType.DMA,
        pltpu.SemaphoreType.DMA,
        pltpu.SemaphoreType.DMA,
    ],
    compiler_params=_SC_PARAMS,
)
def _sc_pool(x_hbm, pidx_hbm, pooled_hbm,
             idx0, idx1, buf0, buf1, ob0, ob1, gs0, gs1, ss0, ss1):
    idxs, bufs, obs = [idx0, idx1], [buf0, buf1], [ob0, ob1]
    gsems, ssems = [gs0, gs1], [ss0, ss1]

    def run(base0, nchunk):
        pltpu.sync_copy(pidx_hbm.at[pl.ds(base0 * K, CHUNK_P * K)], idxs[0])
        gd = [pltpu.async_copy(x_hbm.at[idxs[0]], bufs[0], gsems[0]), None]
        sd = [None, None]
        for c in range(nchunk):
            b, nb = c & 1, (c + 1) & 1
            if c + 1 < nchunk:
                nbase = base0 + (c + 1) * CHUNK_P
                pltpu.sync_copy(pidx_hbm.at[pl.ds(nbase * K, CHUNK_P * K)],
                                idxs[nb])
                gd[nb] = pltpu.async_copy(x_hbm.at[idxs[nb]], bufs[nb],
                                          gsems[nb])
            gd[b].wait()
            if sd[b] is not None:
                sd[b].wait()
            buf, ob = bufs[b], obs[b]

            def row(i, carry):
                for h in range(C_IN // 16):
                    sl = pl.ds(h * 16, 16)
                    s = buf[i * K, sl]
                    for k in range(1, K):
                        s = s + buf[i * K + k, sl]
                    ob[i, sl] = s * (1.0 / K)
                return carry

            lax.fori_loop(0, CHUNK_P, row, 0)
            sd[b] = pltpu.async_copy(
                obs[b], pooled_hbm.at[pl.ds(base0 + c * CHUNK_P, CHUNK_P)],
                ssems[b])
        for d in sd:
            if d is not None:
                d.wait()

    wid = lax.axis_index("s") * NC + lax.axis_index("c")
    run(wid * ROWS_PW, ROWS_PW // CHUNK_P)


def _make_sc_gather(ch, chunk):
    @functools.partial(
        pl.kernel,
        out_type=jax.ShapeDtypeStruct((N_PAD * K, ch), jnp.float32),
        mesh=_MESH,
        scratch_types=[
            pltpu.VMEM((chunk * K,), jnp.int32),
            pltpu.VMEM((chunk * K,), jnp.int32),
            pltpu.VMEM((chunk * K, ch), jnp.float32),
            pltpu.VMEM((chunk * K, ch), jnp.float32),
            pltpu.SemaphoreType.DMA,
            pltpu.SemaphoreType.DMA,
            pltpu.SemaphoreType.DMA,
            pltpu.SemaphoreType.DMA,
        ],
        compiler_params=_SC_PARAMS,
    )
    def _sc_gather(table_hbm, idx_hbm, out_hbm,
                   idx0, idx1, buf0, buf1, gs0, gs1, ss0, ss1):
        idxs, bufs = [idx0, idx1], [buf0, buf1]
        gsems, ssems = [gs0, gs1], [ss0, ss1]

        def run(base0, nchunk):
            pltpu.sync_copy(idx_hbm.at[pl.ds(base0, chunk * K)], idxs[0])
            gd = [pltpu.async_copy(table_hbm.at[idxs[0]], bufs[0], gsems[0]),
                  None]
            sd = [None, None]
            for c in range(nchunk):
                b, nb = c & 1, (c + 1) & 1
                if c + 1 < nchunk:
                    nbase = base0 + (c + 1) * chunk * K
                    pltpu.sync_copy(idx_hbm.at[pl.ds(nbase, chunk * K)],
                                    idxs[nb])
                    if sd[nb] is not None:
                        sd[nb].wait()
                    gd[nb] = pltpu.async_copy(table_hbm.at[idxs[nb]],
                                              bufs[nb], gsems[nb])
                gd[b].wait()
                sd[b] = pltpu.async_copy(
                    bufs[b],
                    out_hbm.at[pl.ds(base0 + c * chunk * K, chunk * K)],
                    ssems[b])
            for d in sd:
                if d is not None:
                    d.wait()

        wid = lax.axis_index("s") * NC + lax.axis_index("c")
        run(wid * ROWS_PW * K, ROWS_PW // chunk)

    return _sc_gather


_sc_gather32 = _make_sc_gather(C_IN, CHUNK_P)
_sc_gather64 = _make_sc_gather(C_OUT, CHUNK_G)


def _tc_mm1_body(g1_ref, w1_ref, b1_ref, h_ref, st_ref):
    i = pl.program_id(0)
    h = lax.dot_general(
        g1_ref[...], w1_ref[...], (((1,), (1,)), ((), ())),
        preferred_element_type=jnp.float32,
    ) + b1_ref[...]
    h_ref[...] = h
    rows = i * TC_TILE + lax.broadcasted_iota(jnp.int32, (TC_TILE, 1), 0)
    hm = jnp.where(rows < N_ROWS, h, 0.0)

    @pl.when(i == 0)
    def _():
        st_ref[...] = jnp.zeros((8, 128), jnp.float32)

    st_ref[0:1, 0:C_OUT] += jnp.sum(hm, axis=0)[None, :]
    st_ref[1:2, 0:C_OUT] += jnp.sum(hm * hm, axis=0)[None, :]


def _tc_mm1(g1, w1, b1):
    return pl.pallas_call(
        _tc_mm1_body,
        grid=(TC_GRID,),
        in_specs=[
            pl.BlockSpec((TC_TILE, K * C_IN), lambda i: (i, 0)),
            pl.BlockSpec((C_OUT, K * C_IN), lambda i: (0, 0)),
            pl.BlockSpec((1, C_OUT), lambda i: (0, 0)),
        ],
        out_specs=[
            pl.BlockSpec((TC_TILE, C_OUT), lambda i: (i, 0)),
            pl.BlockSpec((8, 128), lambda i: (0, 0)),
        ],
        out_shape=[
            jax.ShapeDtypeStruct((N_PAD, C_OUT), jnp.float32),
            jax.ShapeDtypeStruct((8, 128), jnp.float32),
        ],
    )(g1, w1, b1)


def _bn_coeffs(st_ref, gamma_ref, beta_ref, width):
    inv_n = 1.0 / N_ROWS
    mu = st_ref[0:1, 0:width] * inv_n
    var = st_ref[1:2, 0:width] * inv_n - mu * mu
    a = gamma_ref[...] * lax.rsqrt(var + EPS)
    c = beta_ref[...] - a * mu
    return a, c


def _tc_mm2_body(g2_ref, w2_ref, st1_ref, ga1_ref, be1_ref, b2_ref,
                 h2_ref, st2_ref):
    i = pl.program_id(0)
    a, c = _bn_coeffs(st1_ref, ga1_ref, be1_ref, C_OUT)
    acc = jnp.zeros((TC_TILE, C_OUT), jnp.float32)
    for k in range(K):
        z = g2_ref[:, k * C_OUT:(k + 1) * C_OUT] * a + c
        z = jnp.where(z >= 0, z, 0.2 * z)
        acc = acc + lax.dot_general(
            z, w2_ref[:, k * C_OUT:(k + 1) * C_OUT],
            (((1,), (1,)), ((), ())), preferred_element_type=jnp.float32,
        )
    h2 = acc + b2_ref[...]
    h2_ref[...] = h2
    rows = i * TC_TILE + lax.broadcasted_iota(jnp.int32, (TC_TILE, 1), 0)
    hm = jnp.where(rows < N_ROWS, h2, 0.0)

    @pl.when(i == 0)
    def _():
        st2_ref[...] = jnp.zeros((8, 128), jnp.float32)

    st2_ref[0:1, 0:C_OUT] += jnp.sum(hm, axis=0)[None, :]
    st2_ref[1:2, 0:C_OUT] += jnp.sum(hm * hm, axis=0)[None, :]


def _tc_mm2(g2, w2, st1, gamma1, beta1, b2):
    return pl.pallas_call(
        _tc_mm2_body,
        grid=(TC_GRID,),
        in_specs=[
            pl.BlockSpec((TC_TILE, K * C_OUT), lambda i: (i, 0)),
            pl.BlockSpec((C_OUT, K * C_OUT), lambda i: (0, 0)),
            pl.BlockSpec((8, 128), lambda i: (0, 0)),
            pl.BlockSpec((1, C_OUT), lambda i: (0, 0)),
            pl.BlockSpec((1, C_OUT), lambda i: (0, 0)),
            pl.BlockSpec((1, C_OUT), lambda i: (0, 0)),
        ],
        out_specs=[
            pl.BlockSpec((TC_TILE, C_OUT), lambda i: (i, 0)),
            pl.BlockSpec((8, 128), lambda i: (0, 0)),
        ],
        out_shape=[
            jax.ShapeDtypeStruct((N_PAD, C_OUT), jnp.float32),
            jax.ShapeDtypeStruct((8, 128), jnp.float32),
        ],
    )(g2, w2, st1, gamma1, beta1, b2)


def _tc_bn_body(h2_ref, st2_ref, ga2_ref, be2_ref, out_ref):
    a, c = _bn_coeffs(st2_ref, ga2_ref, be2_ref, C_OUT)
    y = h2_ref[...] * a + c
    out_ref[...] = jnp.where(y >= 0, y, 0.2 * y)


def _tc_bn(h2, st2, gamma2, beta2):
    return pl.pallas_call(
        _tc_bn_body,
        grid=(TC_GRID,),
        in_specs=[
            pl.BlockSpec((TC_TILE, C_OUT), lambda i: (i, 0)),
            pl.BlockSpec((8, 128), lambda i: (0, 0)),
            pl.BlockSpec((1, C_OUT), lambda i: (0, 0)),
            pl.BlockSpec((1, C_OUT), lambda i: (0, 0)),
        ],
        out_specs=pl.BlockSpec((TC_TILE, C_OUT), lambda i: (i, 0)),
        out_shape=jax.ShapeDtypeStruct((N_PAD, C_OUT), jnp.float32),
    )(h2, st2, gamma2, beta2)


def kernel(x, neigh_orders, pool_neigh_orders, W1, b1, gamma1, beta1,
           W2, b2, gamma2, beta2):
    pad = (N_PAD - N_ROWS) * K
    pidx = jnp.concatenate(
        [pool_neigh_orders.astype(jnp.int32), jnp.zeros((pad,), jnp.int32)])
    nidx = jnp.concatenate(
        [neigh_orders.astype(jnp.int32), jnp.zeros((pad,), jnp.int32)])

    pooled = _sc_pool(x, pidx)
    g1 = _sc_gather32(pooled, nidx).reshape(N_PAD, K * C_IN)
    h_raw, st1 = _tc_mm1(g1, W1, b1.reshape(1, C_OUT))
    g2 = _sc_gather64(h_raw, nidx).reshape(N_PAD, K * C_OUT)
    h2_raw, st2 = _tc_mm2(g2, W2, st1, gamma1.reshape(1, C_OUT),
                          beta1.reshape(1, C_OUT), b2.reshape(1, C_OUT))
    out = _tc_bn(h2_raw, st2, gamma2.reshape(1, C_OUT),
                 beta2.reshape(1, C_OUT))
    return out[:N_ROWS]
